# Initial kernel scaffold; baseline (speedup 1.0000x reference)
import jax
import jax.numpy as jnp
from jax.experimental import pallas as pl


def kernel(item_ids, table):
    raise NotImplementedError("write your pallas kernel here")


# speedup vs baseline: 1.6831x; 1.6831x over previous
import functools

import jax
import jax.numpy as jnp
from jax import lax
from jax.experimental import pallas as pl
from jax.experimental.pallas import tpu as pltpu
from jax.experimental.pallas import tpu_sc as plsc

EMBED_DIM = 64
BATCH = 16384
HIST = 50

B = BATCH * HIST
NC, NS = 2, 16
NW = NC * NS
PER_W = B // NW
CH = 128
NCHUNK = PER_W // CH


def _make_kernel():
    mesh = plsc.VectorSubcoreMesh(core_axis_name="c", subcore_axis_name="s")

    @functools.partial(
        pl.kernel,
        mesh=mesh,
        out_type=jax.ShapeDtypeStruct((B, EMBED_DIM), jnp.float32),
        scratch_types=[
            pltpu.VMEM((NCHUNK, CH), jnp.int32),
            pltpu.VMEM((CH, EMBED_DIM), jnp.float32),
            pltpu.SemaphoreType.DMA,
        ],
        compiler_params=pltpu.CompilerParams(use_tc_tiling_on_sc=False),
    )
    def k(idx_hbm, table_hbm, out_hbm, idx_v, rows_v, sem):
        wid = lax.axis_index("s") * NC + lax.axis_index("c")
        pltpu.sync_copy(idx_hbm.at[wid], idx_v)

        def body(j, carry):
            pltpu.async_copy(table_hbm.at[idx_v.at[j]], rows_v, sem).wait()
            pltpu.sync_copy(rows_v, out_hbm.at[pl.ds(wid * PER_W + j * CH, CH)])
            return carry

        lax.fori_loop(0, NCHUNK, body, 0)

    return k


_gather_kernel = _make_kernel()


def kernel(item_ids, table):
    idx = item_ids.reshape(NW, NCHUNK, CH).astype(jnp.int32)
    out = _gather_kernel(idx, table)
    return out.reshape(BATCH, HIST, EMBED_DIM)


# speedup vs baseline: 1.8756x; 1.1143x over previous
import functools

import jax
import jax.numpy as jnp
from jax import lax
from jax.experimental import pallas as pl
from jax.experimental.pallas import tpu as pltpu
from jax.experimental.pallas import tpu_sc as plsc

EMBED_DIM = 64
BATCH = 16384
HIST = 50

B = BATCH * HIST
NC, NS = 2, 16
NW = NC * NS
PER_W = B // NW
CH = 128
NCHUNK = PER_W // CH
NBUF = 4
NMACRO = NCHUNK // NBUF


def _make_kernel():
    mesh = plsc.VectorSubcoreMesh(core_axis_name="c", subcore_axis_name="s")

    @functools.partial(
        pl.kernel,
        mesh=mesh,
        out_type=jax.ShapeDtypeStruct((B, EMBED_DIM), jnp.float32),
        scratch_types=[
            pltpu.VMEM((NCHUNK, CH), jnp.int32),
            pltpu.VMEM((NBUF, CH, EMBED_DIM), jnp.float32),
            [pltpu.SemaphoreType.DMA] * NBUF,
            [pltpu.SemaphoreType.DMA] * NBUF,
        ],
        compiler_params=pltpu.CompilerParams(use_tc_tiling_on_sc=False),
    )
    def k(idx_hbm, table_hbm, out_hbm, idx_v, rows_v, gsems, wsems):
        wid = lax.axis_index("s") * NC + lax.axis_index("c")
        base = wid * PER_W
        pltpu.sync_copy(idx_hbm.at[wid], idx_v)

        def gather_start(g, b):
            pltpu.async_copy(table_hbm.at[idx_v.at[g]], rows_v.at[b], gsems[b])

        def gather_wait(g, b):
            pltpu.make_async_copy(
                table_hbm.at[idx_v.at[g]], rows_v.at[b], gsems[b]
            ).wait()

        def write_start(g, b):
            pltpu.async_copy(
                rows_v.at[b], out_hbm.at[pl.ds(base + g * CH, CH)], wsems[b]
            )

        def write_wait(g, b):
            pltpu.make_async_copy(
                rows_v.at[b], out_hbm.at[pl.ds(base + g * CH, CH)], wsems[b]
            ).wait()

        for b in range(NBUF):
            gather_start(b, b)

        for b in range(NBUF):
            gather_wait(b, b)
            write_start(b, b)
            if b >= 1:
                write_wait(b - 1, b - 1)
                gather_start(b - 1 + NBUF, b - 1)

        def macro(kk, carry):
            g0 = kk * NBUF
            for b in range(NBUF):
                g = g0 + b
                gather_wait(g, b)
                write_start(g, b)
                bp = (b - 1) % NBUF
                write_wait(g - 1, bp)
                gather_start(g - 1 + NBUF, bp)
            return carry

        lax.fori_loop(1, NMACRO - 1, macro, 0)

        g0 = NCHUNK - NBUF
        for b in range(NBUF):
            g = g0 + b
            gather_wait(g, b)
            write_start(g, b)
            if b == 0:
                bp = (b - 1) % NBUF
                write_wait(g - 1, bp)
                gather_start(NCHUNK - 1, bp)

        for b in range(NBUF):
            write_wait(g0 + b, b)

    return k


_gather_kernel = _make_kernel()


def kernel(item_ids, table):
    idx = item_ids.reshape(NW, NCHUNK, CH).astype(jnp.int32)
    out = _gather_kernel(idx, table)
    return out.reshape(BATCH, HIST, EMBED_DIM)
